# baseline (speedup 1.0000x reference)
import functools

import jax
import jax.numpy as jnp
from jax import lax
from jax.experimental import pallas as pl
from jax.experimental.pallas import tpu as pltpu
from jax.experimental.pallas import tpu_sc as plsc

_BATCH = 16384
_N_SPANS = 20
_D = 128
_B_TOTAL = _BATCH * _N_SPANS
_NUM_CORES = 2
_NUM_SUBCORES = 16
_NW = _NUM_CORES * _NUM_SUBCORES
_B_PER_W = _B_TOTAL // _NW
_CHUNK = 160
_N_CHUNKS = _B_PER_W // _CHUNK
_NBUF = 4
assert _N_CHUNKS % _NBUF == 0
_TABLE_ROWS = 1000
_REPS = 8


def _sc_gather(table_hbm, idx_hbm, out_hbm, idx_v, *bufs_and_sems):
    bufs = bufs_and_sems[:_NBUF]
    gsems = bufs_and_sems[_NBUF:2 * _NBUF]
    osems = bufs_and_sems[2 * _NBUF:3 * _NBUF]

    wid = lax.axis_index("s") * _NUM_CORES + lax.axis_index("c")
    base = wid * _B_PER_W

    pltpu.sync_copy(idx_hbm.at[pl.ds(base, _B_PER_W)], idx_v)

    def start_gather(c, b):
        pltpu.async_copy(
            table_hbm.at[idx_v.at[pl.ds(c * _CHUNK, _CHUNK)]], bufs[b], gsems[b]
        )

    def wait_gather(c, b):
        pltpu.make_async_copy(
            table_hbm.at[idx_v.at[pl.ds(c * _CHUNK, _CHUNK)]], bufs[b], gsems[b]
        ).wait()

    def start_out(c, b):
        pltpu.async_copy(
            bufs[b], out_hbm.at[pl.ds(base + c * _CHUNK, _CHUNK)], osems[b]
        )

    def wait_out(c, b):
        pltpu.make_async_copy(
            bufs[b], out_hbm.at[pl.ds(base + c * _CHUNK, _CHUNK)], osems[b]
        ).wait()

    start_gather(0, 0)
    start_gather(1, 1)
    for c in (0, 1):
        start_gather(c + 2, (c + 2) % _NBUF)
        wait_gather(c, c % _NBUF)
        start_out(c, c % _NBUF)

    @pl.loop(2, _N_CHUNKS - 2, step=_NBUF)
    def _chunk(c0):
        for b in range(_NBUF):
            c = c0 + b
            wait_out(c - 2, b)
            start_gather(c + 2, b)
            wait_gather(c, (b + 2) % _NBUF)
            start_out(c, (b + 2) % _NBUF)

    for c in (_N_CHUNKS - 2, _N_CHUNKS - 1):
        wait_out(c - 2, (c - 2) % _NBUF)
        wait_gather(c, c % _NBUF)
        start_out(c, c % _NBUF)
    for c in (_N_CHUNKS - 2, _N_CHUNKS - 1):
        wait_out(c, c % _NBUF)


def kernel(span_width, span_width_embeddings):
    idx = span_width.T.reshape(_B_TOTAL)
    rep = (jnp.arange(_B_TOTAL, dtype=jnp.int32) // _B_PER_W) % _REPS
    idx = idx - 1 + rep * _TABLE_ROWS
    table_rep = jnp.tile(span_width_embeddings, (_REPS, 1))
    mesh = plsc.VectorSubcoreMesh(
        core_axis_name="c",
        subcore_axis_name="s",
        num_cores=_NUM_CORES,
        num_subcores=_NUM_SUBCORES,
    )
    run = functools.partial(
        pl.kernel,
        mesh=mesh,
        out_type=jax.ShapeDtypeStruct((_B_TOTAL, _D), jnp.float32),
        scratch_types=(
            [pltpu.VMEM((_B_PER_W,), jnp.int32)]
            + [pltpu.VMEM((_CHUNK, _D), jnp.float32) for _ in range(_NBUF)]
            + [pltpu.SemaphoreType.DMA for _ in range(2 * _NBUF)]
        ),
        compiler_params=pltpu.CompilerParams(use_tc_tiling_on_sc=True),
    )(_sc_gather)
    out = run(table_rep, idx)
    return out.reshape(_N_SPANS, _BATCH, _D).transpose(1, 0, 2)


# speedup vs baseline: 1.0509x; 1.0509x over previous
import functools

import jax
import jax.numpy as jnp
from jax import lax
from jax.experimental import pallas as pl
from jax.experimental.pallas import tpu as pltpu
from jax.experimental.pallas import tpu_sc as plsc

_BATCH = 16384
_N_SPANS = 20
_D = 128
_B_TOTAL = _BATCH * _N_SPANS
_NUM_CORES = 2
_NUM_SUBCORES = 16
_NW = _NUM_CORES * _NUM_SUBCORES
_B_PER_W = _B_TOTAL // _NW
_CHUNK = 160
_N_CHUNKS = _B_PER_W // _CHUNK
_NBUF = 4
assert _N_CHUNKS % _NBUF == 0
_TABLE_ROWS = 1000
_REPS = 16


def _sc_gather(table_hbm, idx_hbm, out_hbm, idx_v, *bufs_and_sems):
    bufs = bufs_and_sems[:_NBUF]
    gsems = bufs_and_sems[_NBUF:2 * _NBUF]
    osems = bufs_and_sems[2 * _NBUF:3 * _NBUF]

    wid = lax.axis_index("s") * _NUM_CORES + lax.axis_index("c")
    base = wid * _B_PER_W

    pltpu.sync_copy(idx_hbm.at[pl.ds(base, _B_PER_W)], idx_v)

    def start_gather(c, b):
        pltpu.async_copy(
            table_hbm.at[idx_v.at[pl.ds(c * _CHUNK, _CHUNK)]], bufs[b], gsems[b]
        )

    def wait_gather(c, b):
        pltpu.make_async_copy(
            table_hbm.at[idx_v.at[pl.ds(c * _CHUNK, _CHUNK)]], bufs[b], gsems[b]
        ).wait()

    def start_out(c, b):
        pltpu.async_copy(
            bufs[b], out_hbm.at[pl.ds(base + c * _CHUNK, _CHUNK)], osems[b]
        )

    def wait_out(c, b):
        pltpu.make_async_copy(
            bufs[b], out_hbm.at[pl.ds(base + c * _CHUNK, _CHUNK)], osems[b]
        ).wait()

    start_gather(0, 0)
    start_gather(1, 1)
    for c in (0, 1):
        start_gather(c + 2, (c + 2) % _NBUF)
        wait_gather(c, c % _NBUF)
        start_out(c, c % _NBUF)

    @pl.loop(2, _N_CHUNKS - 2, step=_NBUF)
    def _chunk(c0):
        for b in range(_NBUF):
            c = c0 + b
            wait_out(c - 2, b)
            start_gather(c + 2, b)
            wait_gather(c, (b + 2) % _NBUF)
            start_out(c, (b + 2) % _NBUF)

    for c in (_N_CHUNKS - 2, _N_CHUNKS - 1):
        wait_out(c - 2, (c - 2) % _NBUF)
        wait_gather(c, c % _NBUF)
        start_out(c, c % _NBUF)
    for c in (_N_CHUNKS - 2, _N_CHUNKS - 1):
        wait_out(c, c % _NBUF)


def kernel(span_width, span_width_embeddings):
    idx = span_width.T.reshape(_B_TOTAL)
    rep = (jnp.arange(_B_TOTAL, dtype=jnp.int32) // _B_PER_W) % _REPS
    idx = idx - 1 + rep * _TABLE_ROWS
    table_rep = jnp.tile(span_width_embeddings, (_REPS, 1))
    mesh = plsc.VectorSubcoreMesh(
        core_axis_name="c",
        subcore_axis_name="s",
        num_cores=_NUM_CORES,
        num_subcores=_NUM_SUBCORES,
    )
    run = functools.partial(
        pl.kernel,
        mesh=mesh,
        out_type=jax.ShapeDtypeStruct((_B_TOTAL, _D), jnp.float32),
        scratch_types=(
            [pltpu.VMEM((_B_PER_W,), jnp.int32)]
            + [pltpu.VMEM((_CHUNK, _D), jnp.float32) for _ in range(_NBUF)]
            + [pltpu.SemaphoreType.DMA for _ in range(2 * _NBUF)]
        ),
        compiler_params=pltpu.CompilerParams(use_tc_tiling_on_sc=True),
    )(_sc_gather)
    out = run(table_rep, idx)
    return out.reshape(_N_SPANS, _BATCH, _D).transpose(1, 0, 2)
